# baseline (speedup 1.0000x reference)
import jax
import jax.numpy as jnp
import numpy as np
from jax.experimental import pallas as pl
from jax.experimental.pallas import tpu as pltpu

_BS = 16
_A = 3
_C = 80
_ATTRS = 5 + _C
_H = 76
_W = 76
_HW = _H * _W
_STRIDE = 8.0
_ANCHORS = np.array([[116.0, 90.0], [156.0, 198.0], [373.0, 326.0]],
                    dtype=np.float32)

_P = 768
_KC = 8
_LAST = _HW - (_KC - 1) * _P
_LFL = 384


def _decode_kernel(x_ref, o_ref, win, carry, head, resid, wsem, csem):
    k = pl.program_id(0)

    n = jax.lax.broadcasted_iota(jnp.int32, (1, 1, _P), 2) + k * _P
    gx = (n % _W).astype(jnp.float32)
    gy = (n // _W).astype(jnp.float32)

    for a in range(_A):
        da = 16 * a
        m = 16 * (a + 1)

        xa = x_ref[:, a]
        t = jnp.transpose(xa, (1, 0, 2))

        s = jax.nn.sigmoid(t)
        row0 = (s[0:1] + gx) * _STRIDE
        row1 = (s[1:2] + gy) * _STRIDE
        row2 = jnp.exp(t[2:3]) * float(_ANCHORS[a, 0])
        row3 = jnp.exp(t[3:4]) * float(_ANCHORS[a, 1])
        r = jnp.concatenate([row0, row1, row2, row3, s[4:]], axis=0)

        if a >= 1:
            @pl.when(k == 0)
            def _():
                head[a - 1] = r[:, :, 0:128]

        if a == 0:
            w = r
        else:
            w = jnp.concatenate(
                [carry[a - 1, :, :, 0:da], r[:, :, 0:_P - da]], axis=2)
            carry[a - 1, :, :, 0:da] = r[:, :, _P - da:_P]

        @pl.when(k == _KC - 1)
        def _():
            resid[a, :, :, 0:m] = r[:, :, _LAST - m:_LAST]

        @pl.when(jnp.logical_and(k >= 1, k <= _KC - 1))
        def _():
            pltpu.make_async_copy(
                win.at[a], o_ref.at[:, :, pl.ds(0, _P)], wsem.at[a]).wait()

        win[a] = w
        dst0 = pl.multiple_of(5760 * a + _P * k, 128)

        @pl.when(k < _KC - 1)
        def _():
            pltpu.make_async_copy(
                win.at[a], o_ref.at[:, :, pl.ds(dst0, _P)],
                wsem.at[a]).start()

        @pl.when(k == _KC - 1)
        def _():
            pltpu.make_async_copy(
                win.at[a, :, :, pl.ds(0, _LFL)],
                o_ref.at[:, :, pl.ds(dst0, _LFL)], wsem.at[a]).start()

    @pl.when(k == _KC - 1)
    def _():
        for a in range(_A):
            pltpu.make_async_copy(
                win.at[a, :, :, pl.ds(0, _LFL)],
                o_ref.at[:, :, pl.ds(0, _LFL)], wsem.at[a]).wait()
        for a in range(_A):
            m = 16 * (a + 1)
            if a < _A - 1:
                tail = head[a, :, :, 0:128 - m]
            else:
                tail = jnp.zeros((_ATTRS, _BS, 128 - m), jnp.float32)
            win[a, :, :, 0:128] = jnp.concatenate(
                [resid[a, :, :, 0:m], tail], axis=2)
            e = pl.multiple_of(5760 * a + _P * k + _LFL, 128)
            pltpu.make_async_copy(
                win.at[a, :, :, pl.ds(0, 128)],
                o_ref.at[:, :, pl.ds(e, 128)], csem.at[a]).start()
        for a in range(_A):
            pltpu.make_async_copy(
                win.at[a, :, :, pl.ds(0, 128)],
                o_ref.at[:, :, pl.ds(0, 128)], csem.at[a]).wait()


def kernel(input):
    x2 = input.reshape(_BS, _A, _ATTRS, _HW)
    out = pl.pallas_call(
        _decode_kernel,
        grid=(_KC,),
        in_specs=[
            pl.BlockSpec((_BS, _A, _ATTRS, _P), lambda k: (0, 0, 0, k)),
        ],
        out_specs=pl.BlockSpec(memory_space=pltpu.HBM),
        out_shape=jax.ShapeDtypeStruct((_ATTRS, _BS, _A * _HW), jnp.float32),
        scratch_shapes=[
            pltpu.VMEM((_A, _ATTRS, _BS, _P), jnp.float32),
            pltpu.VMEM((_A - 1, _ATTRS, _BS, 32), jnp.float32),
            pltpu.VMEM((_A - 1, _ATTRS, _BS, 128), jnp.float32),
            pltpu.VMEM((_A, _ATTRS, _BS, 48), jnp.float32),
            pltpu.SemaphoreType.DMA((_A,)),
            pltpu.SemaphoreType.DMA((_A,)),
        ],
    )(x2)
    return jnp.transpose(out, (1, 2, 0))
